# baseline (speedup 1.0000x reference)
import math

import jax
import jax.numpy as jnp
from jax.experimental import pallas as pl
from jax.experimental.pallas import tpu as pltpu
from jax.experimental.pallas import tpu_sc as plsc

B, CIN, H, W = 4, 384, 32, 32
P = H * W
N = B * P
CE = 256
K = 8192
NCHUNK = 2048
NUM_CHUNKS = K // NCHUNK

SC_CORES = 2
SC_SUBCORES = 16
NW = SC_CORES * SC_SUBCORES
ROWS_PER_W = N // NW
LANES = 16


def _tc_assign_body(x_ref, wp_ref, bp_ref, e_ref,
                    qidx_ref, zsq_ref, dmin_ref, he2_ref):
    b = pl.program_id(0)

    @pl.when(b == 0)
    def _():
        for c in range(NUM_CHUNKS):
            e = e_ref[c * NCHUNK:(c + 1) * NCHUNK, :]
            he2_ref[c * NCHUNK:(c + 1) * NCHUNK, :] = (
                0.5 * jnp.sum(e * e, axis=1, keepdims=True))

    yt = jax.lax.dot_general(
        wp_ref[...], x_ref[0], (((1,), (0,)), ((), ())),
        preferred_element_type=jnp.float32) + bp_ref[...]
    zsq_ref[...] = jnp.sum(yt * yt).reshape(1, 1, 1)

    best_val = jnp.full((1, P), jnp.inf, jnp.float32)
    best_idx = jnp.zeros((1, P), jnp.float32)
    for c in range(NUM_CHUNKS):
        e = e_ref[c * NCHUNK:(c + 1) * NCHUNK, :]
        s = jax.lax.dot_general(
            e, yt, (((1,), (0,)), ((), ())),
            preferred_element_type=jnp.float32)
        d = he2_ref[c * NCHUNK:(c + 1) * NCHUNK, :] - s
        bmin = jnp.min(d, axis=0, keepdims=True)
        rows = jax.lax.broadcasted_iota(
            jnp.int32, (NCHUNK, P), 0).astype(jnp.float32)
        bidx = jnp.min(jnp.where(d == bmin, rows, jnp.float32(3e38)),
                       axis=0, keepdims=True) + float(c * NCHUNK)
        upd = bmin < best_val
        best_val = jnp.where(upd, bmin, best_val)
        best_idx = jnp.where(upd, bidx, best_idx)
    qidx_ref[...] = best_idx.astype(jnp.int32).reshape(1, 8, 128)
    dmin_ref[...] = jnp.sum(best_val).reshape(1, 1, 1)


def _tc_assign(x3, wp, bpc, embed):
    return pl.pallas_call(
        _tc_assign_body,
        grid=(B,),
        in_specs=[
            pl.BlockSpec((1, CIN, P), lambda b: (b, 0, 0)),
            pl.BlockSpec((CE, CIN), lambda b: (0, 0)),
            pl.BlockSpec((CE, 1), lambda b: (0, 0)),
            pl.BlockSpec((K, CE), lambda b: (0, 0)),
        ],
        out_specs=[
            pl.BlockSpec((1, 8, 128), lambda b: (b, 0, 0)),
            pl.BlockSpec((1, 1, 1), lambda b: (b, 0, 0)),
            pl.BlockSpec((1, 1, 1), lambda b: (b, 0, 0)),
        ],
        out_shape=[
            jax.ShapeDtypeStruct((B, 8, 128), jnp.int32),
            jax.ShapeDtypeStruct((B, 1, 1), jnp.float32),
            jax.ShapeDtypeStruct((B, 1, 1), jnp.float32),
        ],
        scratch_shapes=[pltpu.VMEM((K, 1), jnp.float32)],
        compiler_params=pltpu.CompilerParams(
            dimension_semantics=("arbitrary",)),
    )(x3, wp, bpc, embed)


def _sc_gather_body(embed_hbm, idx_hbm, zq_hbm, phist_hbm,
                    idx_v, rows_v, hist_v, sem, sem_out):
    wid = jax.lax.axis_index("s") * SC_CORES + jax.lax.axis_index("c")
    base = wid * ROWS_PER_W

    pltpu.sync_copy(idx_hbm.at[pl.ds(base, ROWS_PER_W)], idx_v)
    gather_cp = pltpu.async_copy(embed_hbm.at[idx_v], rows_v, sem)

    def zero_body(i, carry):
        hist_v[pl.ds(i * LANES, LANES)] = jnp.zeros((LANES,), jnp.float32)
        return carry

    jax.lax.fori_loop(0, K // LANES, zero_body, 0, unroll=16)

    gather_cp.wait()
    out_cp = pltpu.async_copy(rows_v, zq_hbm.at[pl.ds(base, ROWS_PER_W)],
                              sem_out)

    ones = jnp.ones((LANES,), jnp.float32)
    for k in range(ROWS_PER_W // LANES):
        idx_chunk = idx_v[pl.ds(k * LANES, LANES)]
        plsc.addupdate_scatter(hist_v, [idx_chunk], ones)
    pltpu.sync_copy(hist_v, phist_hbm.at[wid])
    out_cp.wait()


def _sc_gather(embed, qidx):
    mesh = plsc.VectorSubcoreMesh(core_axis_name="c", subcore_axis_name="s")
    return pl.kernel(
        _sc_gather_body,
        mesh=mesh,
        out_type=[
            jax.ShapeDtypeStruct((N, CE), jnp.float32),
            jax.ShapeDtypeStruct((NW, K), jnp.float32),
        ],
        scratch_types=[
            pltpu.VMEM((ROWS_PER_W,), jnp.int32),
            pltpu.VMEM((ROWS_PER_W, CE), jnp.float32),
            pltpu.VMEM((K,), jnp.float32),
            pltpu.SemaphoreType.DMA,
            pltpu.SemaphoreType.DMA,
        ],
        compiler_params=pltpu.CompilerParams(needs_layout_passes=False),
    )(embed, qidx)


def _tc_epilogue_body(zsq_ref, dmin_ref, phist_ref,
                      loss_ref, lp_ref, kld_ref):
    sq_total = 2.0 * jnp.sum(dmin_ref[...]) + jnp.sum(zsq_ref[...])
    loss_ref[...] = (1.25 * (sq_total * (1.0 / float(N * CE)))).reshape(1, 1)
    hist = jnp.sum(phist_ref[...], axis=0, keepdims=True)
    p = hist * (1.0 / float(N))
    lp_ref[...] = (-jnp.sum(p * jnp.log(p + 1e-10))).reshape(1, 1)
    kld_ref[...] = jnp.full((B, 1), math.log(K) * float(P), jnp.float32)


def _tc_epilogue(zsq, dmin, phist):
    return pl.pallas_call(
        _tc_epilogue_body,
        out_shape=[
            jax.ShapeDtypeStruct((1, 1), jnp.float32),
            jax.ShapeDtypeStruct((1, 1), jnp.float32),
            jax.ShapeDtypeStruct((B, 1), jnp.float32),
        ],
    )(zsq, dmin, phist)


def kernel(inputs, Wp, bp, embed):
    x3 = inputs.reshape(B, CIN, P)
    qidx, zsq, dmin = _tc_assign(x3, Wp, bp.reshape(CE, 1), embed)
    zq, phist = _sc_gather(embed, qidx.reshape(N))
    loss, lp, kldiv = _tc_epilogue(zsq, dmin, phist)
    z_q = zq.reshape(B, H, W, CE).transpose(0, 3, 1, 2)
    return (z_q, loss[0, 0], kldiv, lp[0, 0])


# speedup vs baseline: 1.0468x; 1.0468x over previous
import math

import jax
import jax.numpy as jnp
from jax.experimental import pallas as pl
from jax.experimental.pallas import tpu as pltpu
from jax.experimental.pallas import tpu_sc as plsc

B, CIN, H, W = 4, 384, 32, 32
P = H * W
N = B * P
CE = 256
K = 8192
NCHUNK = 512
NUM_CHUNKS = K // NCHUNK

SC_CORES = 2
SC_SUBCORES = 16
NW = SC_CORES * SC_SUBCORES
ROWS_PER_W = N // NW
LANES = 16


def _tc_assign_body(x_ref, wp_ref, bp_ref, e_ref,
                    qidx_ref, zsq_ref, dmin_ref, he2_ref):
    b = pl.program_id(0)

    @pl.when(b == 0)
    def _():
        for c in range(NUM_CHUNKS):
            e = e_ref[c * NCHUNK:(c + 1) * NCHUNK, :]
            he2_ref[c * NCHUNK:(c + 1) * NCHUNK, :] = (
                0.5 * jnp.sum(e * e, axis=1, keepdims=True))

    yt = jax.lax.dot_general(
        wp_ref[...], x_ref[0], (((1,), (0,)), ((), ())),
        preferred_element_type=jnp.float32) + bp_ref[...]
    zsq_ref[...] = jnp.sum(yt * yt).reshape(1, 1, 1)

    best_val = jnp.full((1, P), jnp.inf, jnp.float32)
    best_idx = jnp.zeros((1, P), jnp.float32)
    for c in range(NUM_CHUNKS):
        e = e_ref[c * NCHUNK:(c + 1) * NCHUNK, :]
        s = jax.lax.dot_general(
            e, yt, (((1,), (0,)), ((), ())),
            preferred_element_type=jnp.float32)
        d = he2_ref[c * NCHUNK:(c + 1) * NCHUNK, :] - s
        bmin = jnp.min(d, axis=0, keepdims=True)
        rows = jax.lax.broadcasted_iota(
            jnp.int32, (NCHUNK, P), 0).astype(jnp.float32)
        bidx = jnp.min(jnp.where(d == bmin, rows, jnp.float32(3e38)),
                       axis=0, keepdims=True) + float(c * NCHUNK)
        upd = bmin < best_val
        best_val = jnp.where(upd, bmin, best_val)
        best_idx = jnp.where(upd, bidx, best_idx)
    qidx_ref[...] = best_idx.astype(jnp.int32).reshape(1, 8, 128)
    dmin_ref[...] = jnp.sum(best_val).reshape(1, 1, 1)


def _tc_assign(x3, wp, bpc, embed):
    return pl.pallas_call(
        _tc_assign_body,
        grid=(B,),
        in_specs=[
            pl.BlockSpec((1, CIN, P), lambda b: (b, 0, 0)),
            pl.BlockSpec((CE, CIN), lambda b: (0, 0)),
            pl.BlockSpec((CE, 1), lambda b: (0, 0)),
            pl.BlockSpec((K, CE), lambda b: (0, 0)),
        ],
        out_specs=[
            pl.BlockSpec((1, 8, 128), lambda b: (b, 0, 0)),
            pl.BlockSpec((1, 1, 1), lambda b: (b, 0, 0)),
            pl.BlockSpec((1, 1, 1), lambda b: (b, 0, 0)),
        ],
        out_shape=[
            jax.ShapeDtypeStruct((B, 8, 128), jnp.int32),
            jax.ShapeDtypeStruct((B, 1, 1), jnp.float32),
            jax.ShapeDtypeStruct((B, 1, 1), jnp.float32),
        ],
        scratch_shapes=[pltpu.VMEM((K, 1), jnp.float32)],
        compiler_params=pltpu.CompilerParams(
            dimension_semantics=("arbitrary",)),
    )(x3, wp, bpc, embed)


def _sc_gather_body(embed_hbm, idx_hbm, zq_hbm, phist_hbm,
                    idx_v, rows_v, hist_v, sem, sem_out):
    wid = jax.lax.axis_index("s") * SC_CORES + jax.lax.axis_index("c")
    base = wid * ROWS_PER_W

    pltpu.sync_copy(idx_hbm.at[pl.ds(base, ROWS_PER_W)], idx_v)
    gather_cp = pltpu.async_copy(embed_hbm.at[idx_v], rows_v, sem)

    def zero_body(i, carry):
        hist_v[pl.ds(i * LANES, LANES)] = jnp.zeros((LANES,), jnp.float32)
        return carry

    jax.lax.fori_loop(0, K // LANES, zero_body, 0, unroll=16)

    gather_cp.wait()
    out_cp = pltpu.async_copy(rows_v, zq_hbm.at[pl.ds(base, ROWS_PER_W)],
                              sem_out)

    ones = jnp.ones((LANES,), jnp.float32)
    for k in range(ROWS_PER_W // LANES):
        idx_chunk = idx_v[pl.ds(k * LANES, LANES)]
        plsc.addupdate_scatter(hist_v, [idx_chunk], ones)
    pltpu.sync_copy(hist_v, phist_hbm.at[wid])
    out_cp.wait()


def _sc_gather(embed, qidx):
    mesh = plsc.VectorSubcoreMesh(core_axis_name="c", subcore_axis_name="s")
    return pl.kernel(
        _sc_gather_body,
        mesh=mesh,
        out_type=[
            jax.ShapeDtypeStruct((N, CE), jnp.float32),
            jax.ShapeDtypeStruct((NW, K), jnp.float32),
        ],
        scratch_types=[
            pltpu.VMEM((ROWS_PER_W,), jnp.int32),
            pltpu.VMEM((ROWS_PER_W, CE), jnp.float32),
            pltpu.VMEM((K,), jnp.float32),
            pltpu.SemaphoreType.DMA,
            pltpu.SemaphoreType.DMA,
        ],
        compiler_params=pltpu.CompilerParams(needs_layout_passes=False),
    )(embed, qidx)


def _tc_epilogue_body(zsq_ref, dmin_ref, phist_ref,
                      loss_ref, lp_ref, kld_ref):
    sq_total = 2.0 * jnp.sum(dmin_ref[...]) + jnp.sum(zsq_ref[...])
    loss_ref[...] = (1.25 * (sq_total * (1.0 / float(N * CE)))).reshape(1, 1)
    hist = jnp.sum(phist_ref[...], axis=0, keepdims=True)
    p = hist * (1.0 / float(N))
    lp_ref[...] = (-jnp.sum(p * jnp.log(p + 1e-10))).reshape(1, 1)
    kld_ref[...] = jnp.full((B, 1), math.log(K) * float(P), jnp.float32)


def _tc_epilogue(zsq, dmin, phist):
    return pl.pallas_call(
        _tc_epilogue_body,
        out_shape=[
            jax.ShapeDtypeStruct((1, 1), jnp.float32),
            jax.ShapeDtypeStruct((1, 1), jnp.float32),
            jax.ShapeDtypeStruct((B, 1), jnp.float32),
        ],
    )(zsq, dmin, phist)


def kernel(inputs, Wp, bp, embed):
    x3 = inputs.reshape(B, CIN, P)
    qidx, zsq, dmin = _tc_assign(x3, Wp, bp.reshape(CE, 1), embed)
    zq, phist = _sc_gather(embed, qidx.reshape(N))
    loss, lp, kldiv = _tc_epilogue(zsq, dmin, phist)
    z_q = zq.reshape(B, H, W, CE).transpose(0, 3, 1, 2)
    return (z_q, loss[0, 0], kldiv, lp[0, 0])
